# baseline (speedup 1.0000x reference)
import functools

import jax
import jax.numpy as jnp
from jax import lax
from jax.experimental import pallas as pl
from jax.experimental.pallas import tpu as pltpu
from jax.experimental.pallas import tpu_sc as plsc

Q = 512
K = 100000
D = 256
BK = 2048
KP = ((K + BK - 1) // BK) * BK

NC = 2
T = 16
L = 16
CH = KP // T
VL = 32
S2 = CH // VL
HALF = CH // 2
R = 256
NPASS = 4
ROWS_PER_CORE = Q // NC


def _l2n(x):
    n = jnp.sqrt(jnp.sum(x * x, axis=1, keepdims=True))
    return x / jnp.maximum(n, 1e-12)


def _matmul_body(state_ref, mem_ref, w_ref, key_ref):
    i = pl.program_id(0)
    w = jax.lax.dot_general(
        state_ref[...], mem_ref[...], (((1,), (1,)), ((), ())),
        preferred_element_type=jnp.float32,
    )
    w_ref[...] = w
    b = jax.lax.bitcast_convert_type(w, jnp.int32)
    m = jnp.right_shift(b, 31)
    key = jnp.bitwise_xor(b, jnp.bitwise_and(jnp.bitwise_xor(m, -1), 0x7FFFFFFF))
    col = i * BK + jax.lax.broadcasted_iota(jnp.int32, (Q, BK), 1)
    key_ref[...] = jnp.where(col < K, key, -1)


def _cosine_matmul(sn, mn_p):
    return pl.pallas_call(
        _matmul_body,
        grid=(KP // BK,),
        in_specs=[
            pl.BlockSpec((Q, D), lambda i: (0, 0)),
            pl.BlockSpec((BK, D), lambda i: (i, 0)),
        ],
        out_specs=[
            pl.BlockSpec((Q, BK), lambda i: (0, i)),
            pl.BlockSpec((Q, BK), lambda i: (0, i)),
        ],
        out_shape=[
            jax.ShapeDtypeStruct((Q, KP), jnp.float32),
            jax.ShapeDtypeStruct((Q, KP), jnp.int32),
        ],
    )(sn, mn_p)


def _srl(x, n):
    return lax.shift_right_logical(x, lax.full_like(x, n))


_sort_mesh = plsc.VectorSubcoreMesh(
    core_axis_name="c", subcore_axis_name="s", num_cores=NC, num_subcores=T
)


@functools.partial(
    pl.kernel,
    out_type=jax.ShapeDtypeStruct((Q * KP,), jnp.int32),
    mesh=_sort_mesh,
    compiler_params=pltpu.CompilerParams(needs_layout_passes=False),
    scratch_types=[
        pltpu.VMEM((CH,), jnp.int32),
        pltpu.VMEM((CH,), jnp.int32),
        pltpu.VMEM((CH,), jnp.int32),
        pltpu.VMEM((CH,), jnp.int32),
        pltpu.VMEM((R * L,), jnp.int32),
        pltpu.VMEM((R * L,), jnp.int32),
        pltpu.VMEM((R,), jnp.int32),
        pltpu.VMEM((T, R), jnp.int32),
        pltpu.VMEM_SHARED((KP,), jnp.int32),
        pltpu.VMEM_SHARED((T, R), jnp.int32),
        pltpu.SemaphoreType.DMA,
    ],
)
def _radix_argsort(keys_hbm, out_hbm, cur_k, cur_i, const_i, dest, histE,
                   histO, mytot, alltot, idxB, totb, sem):
    c = lax.axis_index("c")
    t = lax.axis_index("s")
    iota = lax.iota(jnp.int32, 16)
    lane_base2 = iota * S2
    ones = jnp.ones((16,), jnp.int32)
    chunk_base = t * CH

    def init_const(j, _):
        const_i[pl.ds(j * 16, 16)] = chunk_base + j * 16 + iota
        return 0
    lax.fori_loop(0, CH // 16, init_const, 0)

    def row_body(ri, _):
        row = c * ROWS_PER_CORE + ri
        rbase = row * KP
        pltpu.sync_copy(keys_hbm.at[pl.ds(rbase + chunk_base, CH)], cur_k)

        for p in range(NPASS):
            shift = 8 * p

            def zero_hist(j, _):
                z = jnp.zeros((16,), jnp.int32)
                histE[pl.ds(j * 16, 16)] = z
                histO[pl.ds(j * 16, 16)] = z
                return 0
            lax.fori_loop(0, R, zero_hist, 0)

            def hist_step(i, _):
                idxA = [lane_base2 + (i * 4 + u) for u in range(4)]
                idxB_ = [HALF + v for v in idxA]
                kA = [plsc.load_gather(cur_k, [v]) for v in idxA]
                kB = [plsc.load_gather(cur_k, [v]) for v in idxB_]
                for u in range(4):
                    dA = jnp.bitwise_and(_srl(kA[u], shift), R - 1)
                    plsc.addupdate_scatter(histE, [dA * 16 + iota], ones)
                    dB = jnp.bitwise_and(_srl(kB[u], shift), R - 1)
                    plsc.addupdate_scatter(histO, [dB * 16 + iota], ones)
                return 0
            lax.fori_loop(0, S2 // 4, hist_step, 0, unroll=2)

            def tot_group(g, _):
                base_slots = (g * 16 + iota) * 16
                acc = jnp.zeros((16,), jnp.int32)
                for l in range(16):
                    acc = acc + plsc.load_gather(histE, [base_slots + l])
                    acc = acc + plsc.load_gather(histO, [base_slots + l])
                mytot[pl.ds(g * 16, 16)] = acc
                return 0
            lax.fori_loop(0, R // 16, tot_group, 0, unroll=2)
            pltpu.sync_copy(mytot, totb.at[t])
            plsc.subcore_barrier()

            pltpu.sync_copy(totb, alltot)
            zero16 = jnp.zeros((16,), jnp.int32)
            tvec = lax.full_like(iota, 0) + t

            def scan_group(g, carry):
                gv = zero16
                mv = zero16
                for tp in range(T):
                    v = alltot[tp, pl.ds(g * 16, 16)]
                    gv = gv + v
                    mv = mv + jnp.where(lax.full_like(iota, tp) < tvec, v, zero16)
                cs = plsc.cumsum(gv)
                acc = (carry + mv) + (cs - gv)
                slots0 = (g * 16 + iota) * 16
                for l in range(L):
                    cv = plsc.load_gather(histE, [slots0 + l])
                    plsc.store_scatter(histE, [slots0 + l], acc)
                    acc = acc + cv
                for l in range(L):
                    cv = plsc.load_gather(histO, [slots0 + l])
                    plsc.store_scatter(histO, [slots0 + l], acc)
                    acc = acc + cv
                return carry + cs[15]
            lax.fori_loop(0, R // 16, scan_group, jnp.int32(0))

            def rank_step(i, _):
                idxA = [lane_base2 + (i * 4 + u) for u in range(4)]
                idxB_ = [HALF + v for v in idxA]
                kA = [plsc.load_gather(cur_k, [v]) for v in idxA]
                kB = [plsc.load_gather(cur_k, [v]) for v in idxB_]
                sA = [jnp.bitwise_and(_srl(k, shift), R - 1) * 16 + iota for k in kA]
                sB = [jnp.bitwise_and(_srl(k, shift), R - 1) * 16 + iota for k in kB]
                pA, pB = [], []
                for u in range(4):
                    pos = plsc.load_gather(histE, [sA[u]])
                    plsc.store_scatter(histE, [sA[u]], pos + 1)
                    pA.append(pos)
                    pos = plsc.load_gather(histO, [sB[u]])
                    plsc.store_scatter(histO, [sB[u]], pos + 1)
                    pB.append(pos)
                for u in range(4):
                    plsc.store_scatter(dest, [idxA[u]], pA[u])
                    plsc.store_scatter(dest, [idxB_[u]], pB[u])
                return 0
            lax.fori_loop(0, S2 // 4, rank_step, 0, unroll=2)

            src_i = const_i if p == 0 else cur_i
            pltpu.sync_copy(src_i, idxB.at[dest])
            plsc.subcore_barrier()

            pltpu.sync_copy(idxB.at[pl.ds(chunk_base, CH)], cur_i)
            if p < NPASS - 1:
                def glob_idx(j, _):
                    dest[pl.ds(j * 16, 16)] = cur_i[pl.ds(j * 16, 16)] + rbase
                    return 0
                lax.fori_loop(0, CH // 16, glob_idx, 0, unroll=8)
                pltpu.async_copy(keys_hbm.at[dest], cur_k, sem).wait()

        pltpu.sync_copy(cur_i, out_hbm.at[pl.ds(rbase + chunk_base, CH)])
        return 0

    lax.fori_loop(0, ROWS_PER_CORE, row_body, 0)


def kernel(state_past, memory_past):
    sn = _l2n(state_past)
    mn = _l2n(memory_past)
    mn_p = jnp.pad(mn, ((0, KP - K), (0, 0)))
    w_pad, keys = _cosine_matmul(sn, mn_p)
    idx = _radix_argsort(keys.reshape(Q * KP)).reshape(Q, KP)
    return (idx[:, :K], w_pad[:, :K])


# speedup vs baseline: 1.4316x; 1.4316x over previous
import functools

import jax
import jax.numpy as jnp
from jax import lax
from jax.experimental import pallas as pl
from jax.experimental.pallas import tpu as pltpu
from jax.experimental.pallas import tpu_sc as plsc

Q = 512
K = 100000
D = 256
BK = 2048
KP = ((K + BK - 1) // BK) * BK

NC = 2
T = 16
L = 16
CH = KP // T
VL = 32
S2 = CH // VL
HALF = CH // 2
R = 256
NPASS = 4
ROWS_PER_CORE = Q // NC


def _l2n(x):
    n = jnp.sqrt(jnp.sum(x * x, axis=1, keepdims=True))
    return x / jnp.maximum(n, 1e-12)


def _matmul_body(state_ref, mem_ref, w_ref, key_ref):
    i = pl.program_id(0)
    w = jax.lax.dot_general(
        state_ref[...], mem_ref[...], (((1,), (1,)), ((), ())),
        preferred_element_type=jnp.float32,
    )
    w_ref[...] = w
    b = jax.lax.bitcast_convert_type(w, jnp.int32)
    m = jnp.right_shift(b, 31)
    key = jnp.bitwise_xor(b, jnp.bitwise_and(jnp.bitwise_xor(m, -1), 0x7FFFFFFF))
    col = i * BK + jax.lax.broadcasted_iota(jnp.int32, (Q, BK), 1)
    key_ref[...] = jnp.where(col < K, key, -1)


def _cosine_matmul(sn, mn_p):
    return pl.pallas_call(
        _matmul_body,
        grid=(KP // BK,),
        in_specs=[
            pl.BlockSpec((Q, D), lambda i: (0, 0)),
            pl.BlockSpec((BK, D), lambda i: (i, 0)),
        ],
        out_specs=[
            pl.BlockSpec((Q, BK), lambda i: (0, i)),
            pl.BlockSpec((Q, BK), lambda i: (0, i)),
        ],
        out_shape=[
            jax.ShapeDtypeStruct((Q, KP), jnp.float32),
            jax.ShapeDtypeStruct((Q, KP), jnp.int32),
        ],
    )(sn, mn_p)


def _srl(x, n):
    return lax.shift_right_logical(x, lax.full_like(x, n))


_sort_mesh = plsc.VectorSubcoreMesh(
    core_axis_name="c", subcore_axis_name="s", num_cores=NC, num_subcores=T
)


@functools.partial(
    pl.kernel,
    out_type=jax.ShapeDtypeStruct((Q * KP,), jnp.int32),
    mesh=_sort_mesh,
    compiler_params=pltpu.CompilerParams(needs_layout_passes=False),
    scratch_types=[
        pltpu.VMEM((CH,), jnp.int32),
        pltpu.VMEM((CH,), jnp.int32),
        pltpu.VMEM((CH,), jnp.int32),
        pltpu.VMEM((CH,), jnp.int32),
        pltpu.VMEM((R * L,), jnp.int32),
        pltpu.VMEM((R * L,), jnp.int32),
        pltpu.VMEM((R,), jnp.int32),
        pltpu.VMEM((T, R), jnp.int32),
        pltpu.VMEM_SHARED((KP,), jnp.int32),
        pltpu.VMEM_SHARED((KP,), jnp.int32),
        pltpu.VMEM_SHARED((T, R), jnp.int32),
        pltpu.SemaphoreType.DMA,
    ],
)
def _radix_argsort(keys_hbm, out_hbm, cur_k, cur_i, const_i, dest, histE,
                   histO, mytot, alltot, keyrow, idxB, totb, sem):
    c = lax.axis_index("c")
    t = lax.axis_index("s")
    iota = lax.iota(jnp.int32, 16)
    lane_base2 = iota * S2
    ones = jnp.ones((16,), jnp.int32)
    chunk_base = t * CH

    def init_const(j, _):
        const_i[pl.ds(j * 16, 16)] = chunk_base + j * 16 + iota
        return 0
    lax.fori_loop(0, CH // 16, init_const, 0)

    def row_body(ri, _):
        row = c * ROWS_PER_CORE + ri
        rbase = row * KP
        lk = pltpu.async_copy(keys_hbm.at[pl.ds(rbase + chunk_base, CH)], cur_k, sem)
        lr = pltpu.async_copy(keys_hbm.at[pl.ds(rbase + chunk_base, CH)],
                              keyrow.at[pl.ds(chunk_base, CH)], sem)
        lk.wait()
        lr.wait()

        for p in range(NPASS):
            shift = 8 * p

            def zero_hist(j, _):
                z = jnp.zeros((16,), jnp.int32)
                histE[pl.ds(j * 16, 16)] = z
                histO[pl.ds(j * 16, 16)] = z
                return 0
            lax.fori_loop(0, R, zero_hist, 0)

            def hist_step(i, _):
                idxA = [lane_base2 + (i * 4 + u) for u in range(4)]
                idxB_ = [HALF + v for v in idxA]
                kA = [plsc.load_gather(cur_k, [v]) for v in idxA]
                kB = [plsc.load_gather(cur_k, [v]) for v in idxB_]
                for u in range(4):
                    dA = jnp.bitwise_and(_srl(kA[u], shift), R - 1)
                    plsc.addupdate_scatter(histE, [dA * 16 + iota], ones)
                    dB = jnp.bitwise_and(_srl(kB[u], shift), R - 1)
                    plsc.addupdate_scatter(histO, [dB * 16 + iota], ones)
                return 0
            lax.fori_loop(0, S2 // 4, hist_step, 0, unroll=4)

            def tot_group(g, _):
                base_slots = (g * 16 + iota) * 16
                acc = jnp.zeros((16,), jnp.int32)
                for l in range(16):
                    acc = acc + plsc.load_gather(histE, [base_slots + l])
                    acc = acc + plsc.load_gather(histO, [base_slots + l])
                mytot[pl.ds(g * 16, 16)] = acc
                return 0
            lax.fori_loop(0, R // 16, tot_group, 0, unroll=2)
            pltpu.sync_copy(mytot, totb.at[t])
            plsc.subcore_barrier()

            pltpu.sync_copy(totb, alltot)
            zero16 = jnp.zeros((16,), jnp.int32)
            tvec = lax.full_like(iota, 0) + t

            def scan_group(g, carry):
                gv = zero16
                mv = zero16
                for tp in range(T):
                    v = alltot[tp, pl.ds(g * 16, 16)]
                    gv = gv + v
                    mv = mv + jnp.where(lax.full_like(iota, tp) < tvec, v, zero16)
                cs = plsc.cumsum(gv)
                acc = (carry + mv) + (cs - gv)
                slots0 = (g * 16 + iota) * 16
                for l in range(L):
                    cv = plsc.load_gather(histE, [slots0 + l])
                    plsc.store_scatter(histE, [slots0 + l], acc)
                    acc = acc + cv
                for l in range(L):
                    cv = plsc.load_gather(histO, [slots0 + l])
                    plsc.store_scatter(histO, [slots0 + l], acc)
                    acc = acc + cv
                return carry + cs[15]
            lax.fori_loop(0, R // 16, scan_group, jnp.int32(0))

            def rank_step(i, _):
                idxA = [lane_base2 + (i * 4 + u) for u in range(4)]
                idxB_ = [HALF + v for v in idxA]
                kA = [plsc.load_gather(cur_k, [v]) for v in idxA]
                kB = [plsc.load_gather(cur_k, [v]) for v in idxB_]
                sA = [jnp.bitwise_and(_srl(k, shift), R - 1) * 16 + iota for k in kA]
                sB = [jnp.bitwise_and(_srl(k, shift), R - 1) * 16 + iota for k in kB]
                pA, pB = [], []
                for u in range(4):
                    pos = plsc.load_gather(histE, [sA[u]])
                    plsc.store_scatter(histE, [sA[u]], pos + 1)
                    pA.append(pos)
                    pos = plsc.load_gather(histO, [sB[u]])
                    plsc.store_scatter(histO, [sB[u]], pos + 1)
                    pB.append(pos)
                for u in range(4):
                    plsc.store_scatter(dest, [idxA[u]], pA[u])
                    plsc.store_scatter(dest, [idxB_[u]], pB[u])
                return 0
            lax.fori_loop(0, S2 // 4, rank_step, 0, unroll=4)

            src_i = const_i if p == 0 else cur_i
            pltpu.sync_copy(src_i, idxB.at[dest])
            plsc.subcore_barrier()

            pltpu.sync_copy(idxB.at[pl.ds(chunk_base, CH)], cur_i)
            if p < NPASS - 1:
                pltpu.async_copy(keyrow.at[cur_i], cur_k, sem).wait()

        pltpu.sync_copy(cur_i, out_hbm.at[pl.ds(rbase + chunk_base, CH)])
        return 0

    lax.fori_loop(0, ROWS_PER_CORE, row_body, 0)


def kernel(state_past, memory_past):
    sn = _l2n(state_past)
    mn = _l2n(memory_past)
    mn_p = jnp.pad(mn, ((0, KP - K), (0, 0)))
    w_pad, keys = _cosine_matmul(sn, mn_p)
    idx = _radix_argsort(keys.reshape(Q * KP)).reshape(Q, KP)
    return (idx[:, :K], w_pad[:, :K])
